# Initial kernel scaffold; baseline (speedup 1.0000x reference)
import jax
import jax.numpy as jnp
from jax.experimental import pallas as pl


def kernel(coords_a, coords_b, point_idx_a, point_idx_b, feats_a, feats_b, fc_w, fc_b):
    raise NotImplementedError("write your pallas kernel here")


# speedup vs baseline: 9.6953x; 9.6953x over previous
import functools

import jax
import jax.numpy as jnp
from jax import lax
from jax.experimental import pallas as pl
from jax.experimental.pallas import tpu as pltpu
from jax.experimental.pallas import tpu_sc as plsc

NA = 8192
NB = 8192
NPLANES = 112
K = 5

RB = 256
CB = 2048
NRB = NA // RB
NCB = NB // CB
BIG = 1e9

NC = 2
NS = 16
NW = NC * NS
QPW = NA // NW
PPW = QPW * K
CHUNK = 128
NCHUNK = PPW // CHUNK
IDX_ROWS = NA * K // 128


def _top5_body(a_ref, b_ref, fb_ref, wb_ref, idx_ref, tmpd_ref, w_ref, carry_ref):
    c = pl.program_id(1)

    @pl.when(c == 0)
    def _init():
        carry_ref[...] = jnp.full((RB, 128), BIG, jnp.float32)
        logit = jnp.sum(fb_ref[...] * wb_ref[0:1, :], axis=1, keepdims=True)
        logit = logit + wb_ref[1:2, 0:1]
        w_ref[...] = jnp.broadcast_to(jax.nn.sigmoid(logit), (RB, 128))

    a = a_ref[...]
    d2 = jnp.zeros((RB, CB), jnp.float32)
    for dim in range(3):
        diff = a[:, dim:dim + 1] - b_ref[dim:dim + 1, :]
        d2 = d2 + diff * diff
    col = (lax.broadcasted_iota(jnp.int32, (RB, CB), 1) + c * CB).astype(jnp.float32)
    keys = jnp.minimum(d2, 256.0) * 8192.0 + col

    vals = jnp.concatenate([keys, carry_ref[...]], axis=1)
    ms = []
    for _ in range(K):
        m = jnp.min(vals, axis=1, keepdims=True)
        ms.append(m)
        vals = jnp.where(vals == m, BIG, vals)
    pad = jnp.full((RB, 128 - K), BIG, jnp.float32)
    carry_ref[...] = jnp.concatenate(ms + [pad], axis=1)

    @pl.when(c == NCB - 1)
    def _fin():
        key5 = jnp.concatenate(ms, axis=1)
        d2c = jnp.floor(key5 * (1.0 / 8192.0))
        jidx = key5 - d2c * 8192.0
        tmpd = 1.0 - jnp.sqrt(d2c) * (1.0 / 16.0)
        zpad = jnp.zeros((RB, 128 - K), jnp.float32)
        idx_ref[...] = jnp.concatenate([jidx, zpad], axis=1).astype(jnp.int32)
        tmpd_ref[...] = jnp.concatenate([tmpd, zpad], axis=1)


def _phase1(a_pad, b_pad, fb_pad, wb_pad):
    return pl.pallas_call(
        _top5_body,
        grid=(NRB, NCB),
        in_specs=[
            pl.BlockSpec((RB, 128), lambda r, c: (r, 0)),
            pl.BlockSpec((8, CB), lambda r, c: (0, c)),
            pl.BlockSpec((RB, 128), lambda r, c: (r, 0)),
            pl.BlockSpec((8, 128), lambda r, c: (0, 0)),
        ],
        out_specs=[
            pl.BlockSpec((RB, 128), lambda r, c: (r, 0)),
            pl.BlockSpec((RB, 128), lambda r, c: (r, 0)),
            pl.BlockSpec((RB, 128), lambda r, c: (r, 0)),
        ],
        out_shape=[
            jax.ShapeDtypeStruct((NA, 128), jnp.int32),
            jax.ShapeDtypeStruct((NA, 128), jnp.float32),
            jax.ShapeDtypeStruct((NA, 128), jnp.float32),
        ],
        scratch_shapes=[pltpu.VMEM((RB, 128), jnp.float32)],
        compiler_params=pltpu.CompilerParams(
            dimension_semantics=("parallel", "arbitrary"),
        ),
    )(a_pad, b_pad, fb_pad, wb_pad)


def _sc_combine(idx2d, tmpd_flat, w_vec, fb_pad):
    mesh = plsc.VectorSubcoreMesh(core_axis_name="c", subcore_axis_name="s")

    @functools.partial(
        pl.kernel,
        mesh=mesh,
        out_type=jax.ShapeDtypeStruct((NA, NPLANES), jnp.float32),
        scratch_types=[
            pltpu.VMEM((NCHUNK, CHUNK), jnp.int32),
            pltpu.VMEM((PPW,), jnp.float32),
            pltpu.VMEM((PPW,), jnp.float32),
            pltpu.VMEM((NB,), jnp.float32),
            pltpu.VMEM((CHUNK, 128), jnp.float32),
            pltpu.VMEM((CHUNK, 128), jnp.float32),
            pltpu.VMEM((QPW, NPLANES), jnp.float32),
            pltpu.SemaphoreType.DMA,
            pltpu.SemaphoreType.DMA,
        ],
        compiler_params=pltpu.CompilerParams(needs_layout_passes=False),
    )
    def k(idx_hbm, tmpd_hbm, w_hbm, fb_hbm, out_hbm,
          idx_v, tmpd_v, coef_v, w_v, buf0, buf1, agg_v, sem0, sem1):
        wid = lax.axis_index("s") * NC + lax.axis_index("c")
        base_p = wid * PPW
        base_q = wid * QPW

        pltpu.sync_copy(w_hbm, w_v)
        pltpu.sync_copy(idx_hbm.at[wid], idx_v)
        pltpu.sync_copy(tmpd_hbm.at[pl.ds(base_p, PPW)], tmpd_v)

        def coef_body(i, _):
            row = i // (CHUNK // 16)
            off = (i % (CHUNK // 16)) * 16
            iv = idx_v[row, pl.ds(off, 16)]
            wv = plsc.load_gather(w_v, [iv])
            s = pl.ds(i * 16, 16)
            coef_v[s] = tmpd_v[s] * wv
            return 0
        lax.fori_loop(0, PPW // 16, coef_body, 0)

        def zero_body(q, _):
            for c7 in range(NPLANES // 16):
                agg_v[q, pl.ds(c7 * 16, 16)] = jnp.zeros((16,), jnp.float32)
            return 0
        lax.fori_loop(0, QPW, zero_body, 0)

        bufs = (buf0, buf1)
        sems = (sem0, sem1)
        copies = [None, None]
        copies[0] = pltpu.async_copy(fb_hbm.at[idx_v.at[0]], buf0, sem0)
        for chunk in range(NCHUNK):
            cur = chunk % 2
            if chunk + 1 < NCHUNK:
                nxt = (chunk + 1) % 2
                copies[nxt] = pltpu.async_copy(
                    fb_hbm.at[idx_v.at[chunk + 1]], bufs[nxt], sems[nxt])
            copies[cur].wait()
            buf = bufs[cur]

            def acc_body(p2, _):
                p = chunk * CHUNK + p2
                q = p // K
                cf = plsc.load_gather(coef_v, [jnp.broadcast_to(p, (16,))])
                for c7 in range(NPLANES // 16):
                    s = pl.ds(c7 * 16, 16)
                    agg_v[q, s] = agg_v[q, s] + cf * buf[p2, s]
                return 0
            lax.fori_loop(0, CHUNK, acc_body, 0)

        pltpu.sync_copy(agg_v, out_hbm.at[pl.ds(base_q, QPW)])

    return k(idx2d, tmpd_flat, w_vec, fb_pad)


def kernel(coords_a, coords_b, point_idx_a, point_idx_b, feats_a, feats_b, fc_w, fc_b):
    del point_idx_a, point_idx_b
    a_pad = jnp.pad(coords_a.astype(jnp.float32), ((0, 0), (0, 128 - 3)))
    b_pad = jnp.pad(coords_b.astype(jnp.float32).T, ((0, 8 - 3), (0, 0)))
    fb_pad = jnp.pad(feats_b, ((0, 0), (0, 128 - NPLANES)))
    wb_pad = jnp.zeros((8, 128), jnp.float32)
    wb_pad = wb_pad.at[0, :NPLANES].set(fc_w[0])
    wb_pad = wb_pad.at[1, 0].set(fc_b[0])

    idx128, tmpd128, w128 = _phase1(a_pad, b_pad, fb_pad, wb_pad)

    idx2d = idx128[:, :K].reshape(NW, NCHUNK, 128)
    tmpd_flat = tmpd128[:, :K].reshape(-1)
    w_vec = w128[:, 0]

    agg = _sc_combine(idx2d, tmpd_flat, w_vec, fb_pad)
    return jnp.concatenate([feats_a, agg], axis=1)


# speedup vs baseline: 12.2510x; 1.2636x over previous
import functools

import jax
import jax.numpy as jnp
from jax import lax
from jax.experimental import pallas as pl
from jax.experimental.pallas import tpu as pltpu
from jax.experimental.pallas import tpu_sc as plsc

NA = 8192
NB = 8192
NPLANES = 112
K = 5

RB = 256
CB = 2048
NRB = NA // RB
NCB = NB // CB
BIG = 1e9

NC = 2
NS = 16
NW = NC * NS
QPW = NA // NW
PPW = QPW * K
CHUNK = 128
NCHUNK = PPW // CHUNK
IDX_ROWS = NA * K // 128


def _top5_body(a_ref, b_ref, bsq_ref, fb_ref, wb_ref,
               idx_ref, tmpd_ref, w_ref, reg_ref):
    c = pl.program_id(1)

    @pl.when(c == 0)
    def _init():
        reg_ref[...] = jnp.full((K, RB, 128), BIG, jnp.float32)
        logit = jnp.sum(fb_ref[...] * wb_ref[0:1, :], axis=1, keepdims=True)
        logit = logit + wb_ref[1:2, 0:1]
        w_ref[...] = jnp.broadcast_to(jax.nn.sigmoid(logit), (RB, 128))

    a_f32 = a_ref[...].astype(jnp.float32)
    an = 0.25 * jnp.sum(a_f32 * a_f32, axis=1, keepdims=True)
    bsq = bsq_ref[...]
    bn = jnp.sum(bsq * bsq, axis=0, keepdims=True)
    ab2 = jnp.dot(a_ref[...], b_ref[...], preferred_element_type=jnp.float32)
    d2 = (ab2 + an) + bn
    col = (lax.broadcasted_iota(jnp.int32, (RB, CB), 1) + c * CB).astype(jnp.float32)
    keys = jnp.minimum(d2, 256.0) * 8192.0 + col

    regs = [reg_ref[t] for t in range(K)]
    for g in range(CB // 128):
        v = keys[:, g * 128:(g + 1) * 128]
        for t in range(K):
            nr = jnp.minimum(regs[t], v)
            if t < K - 1:
                v = jnp.maximum(regs[t], v)
            regs[t] = nr
    for t in range(K):
        reg_ref[t] = regs[t]

    @pl.when(c == NCB - 1)
    def _fin():
        vals = jnp.concatenate(regs, axis=1)
        ms = []
        for _ in range(K):
            m = jnp.min(vals, axis=1, keepdims=True)
            ms.append(m)
            vals = jnp.where(vals == m, BIG, vals)
        key5 = jnp.concatenate(ms, axis=1)
        d2c = jnp.floor(key5 * (1.0 / 8192.0))
        jidx = key5 - d2c * 8192.0
        tmpd = 1.0 - jnp.sqrt(d2c) * (1.0 / 16.0)
        zpad = jnp.zeros((RB, 128 - K), jnp.float32)
        idx_ref[...] = jnp.concatenate([jidx, zpad], axis=1).astype(jnp.int32)
        tmpd_ref[...] = jnp.concatenate([tmpd, zpad], axis=1)


def _phase1(a_bf, b_bf, b_f32, fb_pad, wb_pad):
    return pl.pallas_call(
        _top5_body,
        grid=(NRB, NCB),
        in_specs=[
            pl.BlockSpec((RB, 128), lambda r, c: (r, 0)),
            pl.BlockSpec((128, CB), lambda r, c: (0, c)),
            pl.BlockSpec((8, CB), lambda r, c: (0, c)),
            pl.BlockSpec((RB, 128), lambda r, c: (r, 0)),
            pl.BlockSpec((8, 128), lambda r, c: (0, 0)),
        ],
        out_specs=[
            pl.BlockSpec((RB, 128), lambda r, c: (r, 0)),
            pl.BlockSpec((RB, 128), lambda r, c: (r, 0)),
            pl.BlockSpec((RB, 128), lambda r, c: (r, 0)),
        ],
        out_shape=[
            jax.ShapeDtypeStruct((NA, 128), jnp.int32),
            jax.ShapeDtypeStruct((NA, 128), jnp.float32),
            jax.ShapeDtypeStruct((NA, 128), jnp.float32),
        ],
        scratch_shapes=[pltpu.VMEM((K, RB, 128), jnp.float32)],
        compiler_params=pltpu.CompilerParams(
            dimension_semantics=("parallel", "arbitrary"),
        ),
    )(a_bf, b_bf, b_f32, fb_pad, wb_pad)


def _sc_combine(idx2d, tmpd_flat, w_vec, fb_pad):
    mesh = plsc.VectorSubcoreMesh(core_axis_name="c", subcore_axis_name="s")

    @functools.partial(
        pl.kernel,
        mesh=mesh,
        out_type=jax.ShapeDtypeStruct((NA, NPLANES), jnp.float32),
        scratch_types=[
            pltpu.VMEM((NCHUNK, CHUNK), jnp.int32),
            pltpu.VMEM((PPW,), jnp.float32),
            pltpu.VMEM((PPW,), jnp.float32),
            pltpu.VMEM((NB,), jnp.float32),
            pltpu.VMEM((CHUNK, 128), jnp.float32),
            pltpu.VMEM((CHUNK, 128), jnp.float32),
            pltpu.VMEM((QPW, NPLANES), jnp.float32),
            pltpu.SemaphoreType.DMA,
            pltpu.SemaphoreType.DMA,
        ],
        compiler_params=pltpu.CompilerParams(needs_layout_passes=False),
    )
    def k(idx_hbm, tmpd_hbm, w_hbm, fb_hbm, out_hbm,
          idx_v, tmpd_v, coef_v, w_v, buf0, buf1, agg_v, sem0, sem1):
        wid = lax.axis_index("s") * NC + lax.axis_index("c")
        base_p = wid * PPW
        base_q = wid * QPW

        pltpu.sync_copy(w_hbm, w_v)
        pltpu.sync_copy(idx_hbm.at[wid], idx_v)
        pltpu.sync_copy(tmpd_hbm.at[pl.ds(base_p, PPW)], tmpd_v)

        def coef_body(i, _):
            row = i // (CHUNK // 16)
            off = (i % (CHUNK // 16)) * 16
            iv = idx_v[row, pl.ds(off, 16)]
            wv = plsc.load_gather(w_v, [iv])
            s = pl.ds(i * 16, 16)
            coef_v[s] = tmpd_v[s] * wv
            return 0
        lax.fori_loop(0, PPW // 16, coef_body, 0)

        def zero_body(q, _):
            for c7 in range(NPLANES // 16):
                agg_v[q, pl.ds(c7 * 16, 16)] = jnp.zeros((16,), jnp.float32)
            return 0
        lax.fori_loop(0, QPW, zero_body, 0)

        bufs = (buf0, buf1)
        sems = (sem0, sem1)
        copies = [None, None]
        copies[0] = pltpu.async_copy(fb_hbm.at[idx_v.at[0]], buf0, sem0)
        for chunk in range(NCHUNK):
            cur = chunk % 2
            if chunk + 1 < NCHUNK:
                nxt = (chunk + 1) % 2
                copies[nxt] = pltpu.async_copy(
                    fb_hbm.at[idx_v.at[chunk + 1]], bufs[nxt], sems[nxt])
            copies[cur].wait()
            buf = bufs[cur]

            def acc_body(p2, _):
                p = chunk * CHUNK + p2
                q = p // K
                cf = plsc.load_gather(coef_v, [jnp.broadcast_to(p, (16,))])
                for c7 in range(NPLANES // 16):
                    s = pl.ds(c7 * 16, 16)
                    agg_v[q, s] = agg_v[q, s] + cf * buf[p2, s]
                return 0
            lax.fori_loop(0, CHUNK, acc_body, 0)

        pltpu.sync_copy(agg_v, out_hbm.at[pl.ds(base_q, QPW)])

    return k(idx2d, tmpd_flat, w_vec, fb_pad)


def kernel(coords_a, coords_b, point_idx_a, point_idx_b, feats_a, feats_b, fc_w, fc_b):
    del point_idx_a, point_idx_b
    a_bf = jnp.pad((-2.0 * coords_a.astype(jnp.float32)).astype(jnp.bfloat16),
                   ((0, 0), (0, 128 - 3)))
    cbt = coords_b.astype(jnp.float32).T
    b_bf = jnp.pad(cbt.astype(jnp.bfloat16), ((0, 128 - 3), (0, 0)))
    b_f32 = jnp.pad(cbt, ((0, 8 - 3), (0, 0)))
    fb_pad = jnp.pad(feats_b, ((0, 0), (0, 128 - NPLANES)))
    wb_pad = jnp.zeros((8, 128), jnp.float32)
    wb_pad = wb_pad.at[0, :NPLANES].set(fc_w[0])
    wb_pad = wb_pad.at[1, 0].set(fc_b[0])

    idx128, tmpd128, w128 = _phase1(a_bf, b_bf, b_f32, fb_pad, wb_pad)

    idx2d = idx128[:, :K].reshape(NW, NCHUNK, 128)
    tmpd_flat = tmpd128[:, :K].reshape(-1)
    w_vec = w128[:, 0]

    agg = _sc_combine(idx2d, tmpd_flat, w_vec, fb_pad)
    return jnp.concatenate([feats_a, agg], axis=1)
